# Initial kernel scaffold; baseline (speedup 1.0000x reference)
import jax
import jax.numpy as jnp
from jax.experimental import pallas as pl


def kernel(inputs):
    raise NotImplementedError("write your pallas kernel here")


# speedup vs baseline: 1.4387x; 1.4387x over previous
import jax
import jax.numpy as jnp
from jax.experimental import pallas as pl

_B, _S, _F = 4, 2048, 1024
_ROWS = _B * _S
_BLK_ROWS = 512
_LANE = 128


def _rev_body(x_ref, j_ref, out_ref, zero_ref):
    out_ref[...] = jax.lax.dot(
        x_ref[...], j_ref[...],
        precision=jax.lax.Precision.HIGHEST,
        preferred_element_type=jnp.float32,
    )
    zero_ref[...] = jnp.zeros_like(zero_ref)


def kernel(inputs):
    x = inputs.reshape(_ROWS, _F)
    jmat = jnp.eye(_LANE, dtype=jnp.float32)[::-1]
    nlb = _F // _LANE
    grid = (_ROWS // _BLK_ROWS, nlb)
    out, zeros = pl.pallas_call(
        _rev_body,
        grid=grid,
        in_specs=[
            pl.BlockSpec((_BLK_ROWS, _LANE), lambda i, j: (i, nlb - 1 - j)),
            pl.BlockSpec((_LANE, _LANE), lambda i, j: (0, 0)),
        ],
        out_specs=[
            pl.BlockSpec((_BLK_ROWS, _LANE), lambda i, j: (i, j)),
            pl.BlockSpec((_BLK_ROWS, _LANE), lambda i, j: (i, j)),
        ],
        out_shape=[
            jax.ShapeDtypeStruct((_ROWS, _F), jnp.float32),
            jax.ShapeDtypeStruct((_ROWS, _F), jnp.float32),
        ],
    )(x, jmat)
    return (out.reshape(_B, _S, _F), zeros.reshape(_B, _S, _F))


# speedup vs baseline: 2.9305x; 2.0369x over previous
import jax
import jax.numpy as jnp
from jax.experimental import pallas as pl

_B, _S, _F = 4, 2048, 1024
_ROWS = _B * _S
_BLK_ROWS = 2048
_LANE = 128


def _rev_body(x_ref, j_ref, out_ref, zero_ref):
    out_ref[...] = jax.lax.dot(
        x_ref[...], j_ref[...],
        precision=jax.lax.Precision.HIGHEST,
        preferred_element_type=jnp.float32,
    )
    zero_ref[...] = jnp.zeros_like(zero_ref)


def kernel(inputs):
    x = inputs.reshape(_ROWS, _F)
    jmat = jnp.eye(_LANE, dtype=jnp.float32)[::-1]
    nlb = _F // _LANE
    grid = (_ROWS // _BLK_ROWS, nlb)
    out, zeros = pl.pallas_call(
        _rev_body,
        grid=grid,
        in_specs=[
            pl.BlockSpec((_BLK_ROWS, _LANE), lambda i, j: (i, nlb - 1 - j)),
            pl.BlockSpec((_LANE, _LANE), lambda i, j: (0, 0)),
        ],
        out_specs=[
            pl.BlockSpec((_BLK_ROWS, _LANE), lambda i, j: (i, j)),
            pl.BlockSpec((_BLK_ROWS, _LANE), lambda i, j: (i, j)),
        ],
        out_shape=[
            jax.ShapeDtypeStruct((_ROWS, _F), jnp.float32),
            jax.ShapeDtypeStruct((_ROWS, _F), jnp.float32),
        ],
    )(x, jmat)
    return (out.reshape(_B, _S, _F), zeros.reshape(_B, _S, _F))


# speedup vs baseline: 3.4959x; 1.1929x over previous
import jax
import jax.numpy as jnp
from jax.experimental import pallas as pl

_B, _S, _F = 4, 2048, 1024
_ROWS = _B * _S
_BLK_ROWS = 4096
_LANE = 128


def _rev_body(x_ref, j_ref, out_ref, zero_ref):
    out_ref[...] = jax.lax.dot(
        x_ref[...], j_ref[...],
        precision=jax.lax.Precision.HIGHEST,
        preferred_element_type=jnp.float32,
    )
    zero_ref[...] = jnp.zeros_like(zero_ref)


def kernel(inputs):
    x = inputs.reshape(_ROWS, _F)
    jmat = jnp.eye(_LANE, dtype=jnp.float32)[::-1]
    nlb = _F // _LANE
    grid = (_ROWS // _BLK_ROWS, nlb)
    out, zeros = pl.pallas_call(
        _rev_body,
        grid=grid,
        in_specs=[
            pl.BlockSpec((_BLK_ROWS, _LANE), lambda i, j: (i, nlb - 1 - j)),
            pl.BlockSpec((_LANE, _LANE), lambda i, j: (0, 0)),
        ],
        out_specs=[
            pl.BlockSpec((_BLK_ROWS, _LANE), lambda i, j: (i, j)),
            pl.BlockSpec((_BLK_ROWS, _LANE), lambda i, j: (i, j)),
        ],
        out_shape=[
            jax.ShapeDtypeStruct((_ROWS, _F), jnp.float32),
            jax.ShapeDtypeStruct((_ROWS, _F), jnp.float32),
        ],
    )(x, jmat)
    return (out.reshape(_B, _S, _F), zeros.reshape(_B, _S, _F))


# speedup vs baseline: 3.7952x; 1.0856x over previous
import jax
import jax.numpy as jnp
from jax.experimental import pallas as pl

_B, _S, _F = 4, 2048, 1024
_ROWS = _B * _S
_BLK_ROWS = 8192
_LANE = 128


def _rev_body(x_ref, j_ref, out_ref, zero_ref):
    out_ref[...] = jax.lax.dot(
        x_ref[...], j_ref[...],
        precision=jax.lax.Precision.HIGHEST,
        preferred_element_type=jnp.float32,
    )
    zero_ref[...] = jnp.zeros_like(zero_ref)


def kernel(inputs):
    x = inputs.reshape(_ROWS, _F)
    jmat = jnp.eye(_LANE, dtype=jnp.float32)[::-1]
    nlb = _F // _LANE
    grid = (_ROWS // _BLK_ROWS, nlb)
    out, zeros = pl.pallas_call(
        _rev_body,
        grid=grid,
        in_specs=[
            pl.BlockSpec((_BLK_ROWS, _LANE), lambda i, j: (i, nlb - 1 - j)),
            pl.BlockSpec((_LANE, _LANE), lambda i, j: (0, 0)),
        ],
        out_specs=[
            pl.BlockSpec((_BLK_ROWS, _LANE), lambda i, j: (i, j)),
            pl.BlockSpec((_BLK_ROWS, _LANE), lambda i, j: (i, j)),
        ],
        out_shape=[
            jax.ShapeDtypeStruct((_ROWS, _F), jnp.float32),
            jax.ShapeDtypeStruct((_ROWS, _F), jnp.float32),
        ],
    )(x, jmat)
    return (out.reshape(_B, _S, _F), zeros.reshape(_B, _S, _F))
